# baseline (speedup 1.0000x reference)
import functools

import jax
import jax.numpy as jnp
from jax import lax
from jax.experimental import pallas as pl
from jax.experimental.pallas import tpu as pltpu
from jax.experimental.pallas import tpu_sc as plsc


def _pack_pair(lo_f32, hi_f32):
    lo_u = lax.bitcast_convert_type(
        lo_f32.astype(jnp.bfloat16).astype(jnp.float32), jnp.uint32)
    hi_u = lax.bitcast_convert_type(
        hi_f32.astype(jnp.bfloat16).astype(jnp.float32), jnp.uint32)
    packed = (lo_u >> 16) | (hi_u & jnp.uint32(0xFFFF0000))
    return lax.bitcast_convert_type(packed, jnp.float32)


def _unpack_pair(packed_f32):
    u = lax.bitcast_convert_type(packed_f32, jnp.uint32)
    lo = lax.bitcast_convert_type(u << 16, jnp.float32)
    hi = lax.bitcast_convert_type(u & jnp.uint32(0xFFFF0000), jnp.float32)
    return lo, hi


def _proj_body(nodes_ref, w_ref, out_ref):
    p = jnp.dot(nodes_ref[...], w_ref[...],
                preferred_element_type=jnp.float32)
    h2 = p.shape[1]
    q = h2 // 4
    pack_s = _pack_pair(p[:, 0 * q:1 * q], p[:, 1 * q:2 * q])
    pack_r = _pack_pair(p[:, 2 * q:3 * q], p[:, 3 * q:4 * q])
    out_ref[...] = jnp.concatenate([pack_s, pack_r], axis=1)


def _project_nodes(nodes, w_sr):
    n, d = nodes.shape
    two_h = w_sr.shape[1]
    bn = 2000
    grid = (n // bn,)
    return pl.pallas_call(
        _proj_body,
        grid=grid,
        in_specs=[
            pl.BlockSpec((bn, d), lambda i: (i, 0)),
            pl.BlockSpec((d, two_h), lambda i: (0, 0)),
        ],
        out_specs=pl.BlockSpec((bn, two_h // 2), lambda i: (i, 0)),
        out_shape=jax.ShapeDtypeStruct((n, two_h // 2), jnp.float32),
    )(nodes, w_sr)


_CH = 40


def _sc_gather(table, idx, h):
    info = plsc.get_sparse_core_info()
    nc, ns = info.num_cores, info.num_subcores
    nw = nc * ns
    total_rows = idx.shape[0]
    rows_per_w = total_rows // nw
    n_ch = rows_per_w // _CH
    dt = table.dtype
    mesh = plsc.VectorSubcoreMesh(core_axis_name="c", subcore_axis_name="s")

    @functools.partial(
        pl.kernel,
        mesh=mesh,
        out_type=jax.ShapeDtypeStruct((total_rows, h), dt),
        scratch_types=[
            pltpu.VMEM((rows_per_w,), jnp.int32),
            pltpu.VMEM((_CH, h), dt),
            pltpu.VMEM((_CH, h), dt),
            pltpu.SemaphoreType.DMA,
            pltpu.SemaphoreType.DMA,
            pltpu.SemaphoreType.DMA,
            pltpu.SemaphoreType.DMA,
        ],
    )
    def k(table_hbm, idx_hbm, out_hbm, idx_v, rows0, rows1, gs0, gs1, os0,
          os1):
        wid = lax.axis_index("s") * nc + lax.axis_index("c")
        rbase = wid * rows_per_w
        pltpu.sync_copy(idx_hbm.at[pl.ds(rbase, rows_per_w)], idx_v)
        rows = (rows0, rows1)
        gsem = (gs0, gs1)
        osem = (os0, os1)

        def gstart(c, b):
            pltpu.async_copy(table_hbm.at[idx_v.at[pl.ds(c * _CH, _CH)]],
                             rows[b], gsem[b])

        def ostart(c, b):
            pltpu.async_copy(rows[b],
                             out_hbm.at[pl.ds(rbase + c * _CH, _CH)],
                             osem[b])

        def owait(b):
            pltpu.make_async_copy(rows[b],
                                  out_hbm.at[pl.ds(rbase, _CH)],
                                  osem[b]).wait()

        def gwait(b):
            pltpu.make_async_copy(table_hbm.at[idx_v.at[pl.ds(0, _CH)]],
                                  rows[b], gsem[b]).wait()

        gstart(0, 0)

        def outer(c0, carry):
            gwait(0)
            @pl.when(c0 > 0)
            def _():
                owait(1)
            gstart(c0 + 1, 1)
            ostart(c0, 0)
            gwait(1)
            owait(0)
            @pl.when(c0 + 2 < n_ch)
            def _():
                gstart(c0 + 2, 0)
            ostart(c0 + 1, 1)
            return carry

        lax.fori_loop(0, n_ch // 2, lambda j, c: outer(j * 2, c), 0)
        owait(1)

    return k(table, idx)


def _mlp_body(sg_ref, rg_ref, ea_ref, g_ref, w1e_ref, w1g_ref, b1_ref,
              w2_ref, b2_ref, out_ref):
    s_lo, s_hi = _unpack_pair(sg_ref[...])
    r_lo, r_hi = _unpack_pair(rg_ref[...])
    nodes_term = jnp.concatenate([s_lo + r_lo, s_hi + r_hi], axis=1)
    acc = nodes_term + jnp.dot(ea_ref[...], w1e_ref[...],
                               preferred_element_type=jnp.float32)
    gterm = jnp.dot(g_ref[...], w1g_ref[...],
                    preferred_element_type=jnp.float32)
    h = jnp.maximum(acc + gterm + b1_ref[...], 0.0)
    out_ref[...] = jnp.dot(h.astype(jnp.bfloat16), w2_ref[...],
                           preferred_element_type=jnp.float32) + b2_ref[...]


def _edge_mlp(gathered, ea, g2d, w1e, w1g, b1, w2, b2):
    e, de = ea.shape
    h = w2.shape[0]
    dout = w2.shape[1]
    be = 800
    nblk = e // be
    return pl.pallas_call(
        _mlp_body,
        grid=(nblk,),
        in_specs=[
            pl.BlockSpec((be, h // 2), lambda i: (i, 0)),
            pl.BlockSpec((be, h // 2), lambda i: (i + nblk, 0)),
            pl.BlockSpec((be, de), lambda i: (i, 0)),
            pl.BlockSpec(g2d.shape, lambda i: (0, 0)),
            pl.BlockSpec(w1e.shape, lambda i: (0, 0)),
            pl.BlockSpec(w1g.shape, lambda i: (0, 0)),
            pl.BlockSpec(b1.shape, lambda i: (0, 0)),
            pl.BlockSpec(w2.shape, lambda i: (0, 0)),
            pl.BlockSpec(b2.shape, lambda i: (0, 0)),
        ],
        out_specs=pl.BlockSpec((be, dout), lambda i: (i, 0)),
        out_shape=jax.ShapeDtypeStruct((e, dout), jnp.float32),
    )(gathered, gathered, ea, g2d, w1e, w1g, b1, w2, b2)


def kernel(node_attributes, edge_index, edge_attributes, global_attributes,
           W1, b1, W2, b2):
    n, d = node_attributes.shape
    e, de = edge_attributes.shape
    dg = global_attributes.shape[0]
    h = W1.shape[1]

    w1e = W1[:de]
    w_sr = jnp.concatenate([W1[de:de + d], W1[de + d:de + 2 * d]], axis=1)
    w1g = W1[de + 2 * d:]

    proj = _project_nodes(node_attributes, w_sr).reshape(2 * n, h // 2)

    idx = jnp.concatenate([2 * edge_index[0], 2 * edge_index[1] + 1])

    gathered = _sc_gather(proj, idx, h // 2)

    return _edge_mlp(gathered, edge_attributes,
                     global_attributes.reshape(1, dg),
                     w1e, w1g, b1.reshape(1, h), W2.astype(jnp.bfloat16),
                     b2.reshape(1, -1))


# speedup vs baseline: 1.0328x; 1.0328x over previous
import functools

import jax
import jax.numpy as jnp
from jax import lax
from jax.experimental import pallas as pl
from jax.experimental.pallas import tpu as pltpu
from jax.experimental.pallas import tpu_sc as plsc


def _pack_pair(lo_f32, hi_f32):
    lo_u = lax.bitcast_convert_type(
        lo_f32.astype(jnp.bfloat16).astype(jnp.float32), jnp.uint32)
    hi_u = lax.bitcast_convert_type(
        hi_f32.astype(jnp.bfloat16).astype(jnp.float32), jnp.uint32)
    packed = (lo_u >> 16) | (hi_u & jnp.uint32(0xFFFF0000))
    return lax.bitcast_convert_type(packed, jnp.float32)


def _unpack_pair(packed_f32):
    u = lax.bitcast_convert_type(packed_f32, jnp.uint32)
    lo = lax.bitcast_convert_type(u << 16, jnp.float32)
    hi = lax.bitcast_convert_type(u & jnp.uint32(0xFFFF0000), jnp.float32)
    return lo, hi


def _proj_body(nodes_ref, w_ref, out_ref):
    p = jnp.dot(nodes_ref[...], w_ref[...],
                preferred_element_type=jnp.float32)
    h2 = p.shape[1]
    q = h2 // 4
    pack_s = _pack_pair(p[:, 0 * q:1 * q], p[:, 1 * q:2 * q])
    pack_r = _pack_pair(p[:, 2 * q:3 * q], p[:, 3 * q:4 * q])
    out_ref[...] = jnp.concatenate([pack_s, pack_r], axis=1)


def _project_nodes(nodes, w_sr):
    n, d = nodes.shape
    two_h = w_sr.shape[1]
    bn = 2000
    grid = (n // bn,)
    return pl.pallas_call(
        _proj_body,
        grid=grid,
        in_specs=[
            pl.BlockSpec((bn, d), lambda i: (i, 0)),
            pl.BlockSpec((d, two_h), lambda i: (0, 0)),
        ],
        out_specs=pl.BlockSpec((bn, two_h // 2), lambda i: (i, 0)),
        out_shape=jax.ShapeDtypeStruct((n, two_h // 2), jnp.float32),
    )(nodes, w_sr)


_CH = 40


def _sc_gather(table, idx, h):
    info = plsc.get_sparse_core_info()
    nc, ns = info.num_cores, info.num_subcores
    nw = nc * ns
    total_rows = idx.shape[0]
    rows_per_w = total_rows // nw
    n_ch = rows_per_w // _CH
    dt = table.dtype
    mesh = plsc.VectorSubcoreMesh(core_axis_name="c", subcore_axis_name="s")

    @functools.partial(
        pl.kernel,
        mesh=mesh,
        out_type=jax.ShapeDtypeStruct((total_rows, h), dt),
        scratch_types=[
            pltpu.VMEM((rows_per_w,), jnp.int32),
            pltpu.VMEM((_CH, h), dt),
            pltpu.VMEM((_CH, h), dt),
            pltpu.SemaphoreType.DMA,
            pltpu.SemaphoreType.DMA,
            pltpu.SemaphoreType.DMA,
            pltpu.SemaphoreType.DMA,
        ],
    )
    def k(table_hbm, idx_hbm, out_hbm, idx_v, rows0, rows1, gs0, gs1, os0,
          os1):
        wid = lax.axis_index("s") * nc + lax.axis_index("c")
        rbase = wid * rows_per_w
        pltpu.sync_copy(idx_hbm.at[pl.ds(rbase, rows_per_w)], idx_v)
        rows = (rows0, rows1)
        gsem = (gs0, gs1)
        osem = (os0, os1)

        def gstart(c, b):
            pltpu.async_copy(table_hbm.at[idx_v.at[pl.ds(c * _CH, _CH)]],
                             rows[b], gsem[b])

        def ostart(c, b):
            pltpu.async_copy(rows[b],
                             out_hbm.at[pl.ds(rbase + c * _CH, _CH)],
                             osem[b])

        def owait(b):
            pltpu.make_async_copy(rows[b],
                                  out_hbm.at[pl.ds(rbase, _CH)],
                                  osem[b]).wait()

        def gwait(b):
            pltpu.make_async_copy(table_hbm.at[idx_v.at[pl.ds(0, _CH)]],
                                  rows[b], gsem[b]).wait()

        gstart(0, 0)

        def outer(c0, carry):
            gwait(0)
            @pl.when(c0 > 0)
            def _():
                owait(1)
            gstart(c0 + 1, 1)
            ostart(c0, 0)
            gwait(1)
            owait(0)
            @pl.when(c0 + 2 < n_ch)
            def _():
                gstart(c0 + 2, 0)
            ostart(c0 + 1, 1)
            return carry

        lax.fori_loop(0, n_ch // 2, lambda j, c: outer(j * 2, c), 0)
        owait(1)

    return k(table, idx)


def _mlp_body(sg_ref, rg_ref, ea_ref, g_ref, w1e_ref, w1g_ref, b1_ref,
              w2_ref, b2_ref, out_ref):
    s_lo, s_hi = _unpack_pair(sg_ref[...])
    r_lo, r_hi = _unpack_pair(rg_ref[...])
    nodes_term = jnp.concatenate([s_lo + r_lo, s_hi + r_hi], axis=1)
    acc = nodes_term + jnp.dot(ea_ref[...], w1e_ref[...],
                               preferred_element_type=jnp.float32)
    gterm = jnp.dot(g_ref[...], w1g_ref[...],
                    preferred_element_type=jnp.float32)
    h = jnp.maximum(acc + gterm + b1_ref[...], 0.0)
    out_ref[...] = jnp.dot(h, w2_ref[...],
                           preferred_element_type=jnp.float32) + b2_ref[...]


def _edge_mlp(gathered, ea, g2d, w1e, w1g, b1, w2, b2):
    e, de = ea.shape
    h = w2.shape[0]
    dout = w2.shape[1]
    be = 800
    nblk = e // be
    return pl.pallas_call(
        _mlp_body,
        grid=(nblk,),
        in_specs=[
            pl.BlockSpec((be, h // 2), lambda i: (i, 0)),
            pl.BlockSpec((be, h // 2), lambda i: (i + nblk, 0)),
            pl.BlockSpec((be, de), lambda i: (i, 0)),
            pl.BlockSpec(g2d.shape, lambda i: (0, 0)),
            pl.BlockSpec(w1e.shape, lambda i: (0, 0)),
            pl.BlockSpec(w1g.shape, lambda i: (0, 0)),
            pl.BlockSpec(b1.shape, lambda i: (0, 0)),
            pl.BlockSpec(w2.shape, lambda i: (0, 0)),
            pl.BlockSpec(b2.shape, lambda i: (0, 0)),
        ],
        out_specs=pl.BlockSpec((be, dout), lambda i: (i, 0)),
        out_shape=jax.ShapeDtypeStruct((e, dout), jnp.float32),
    )(gathered, gathered, ea, g2d, w1e, w1g, b1, w2, b2)


def kernel(node_attributes, edge_index, edge_attributes, global_attributes,
           W1, b1, W2, b2):
    n, d = node_attributes.shape
    e, de = edge_attributes.shape
    dg = global_attributes.shape[0]
    h = W1.shape[1]

    w1e = W1[:de]
    w_sr = jnp.concatenate([W1[de:de + d], W1[de + d:de + 2 * d]], axis=1)
    w1g = W1[de + 2 * d:]

    proj = _project_nodes(node_attributes, w_sr).reshape(2 * n, h // 2)

    nslab = 5
    es = e // nslab
    src2 = 2 * edge_index[0]
    dst2 = 2 * edge_index[1] + 1
    g2d = global_attributes.reshape(1, dg)
    b1r = b1.reshape(1, h)
    b2r = b2.reshape(1, -1)
    outs = []
    for k in range(nslab):
        idx_k = jnp.concatenate([lax.dynamic_slice_in_dim(src2, k * es, es),
                                 lax.dynamic_slice_in_dim(dst2, k * es, es)])
        gathered = _sc_gather(proj, idx_k, h // 2)
        outs.append(_edge_mlp(gathered,
                              lax.dynamic_slice_in_dim(edge_attributes,
                                                       k * es, es),
                              g2d, w1e, w1g, b1r, W2, b2r))
    return jnp.concatenate(outs, axis=0)


# speedup vs baseline: 1.0437x; 1.0106x over previous
import functools

import jax
import jax.numpy as jnp
from jax import lax
from jax.experimental import pallas as pl
from jax.experimental.pallas import tpu as pltpu
from jax.experimental.pallas import tpu_sc as plsc


def _pack_pair(lo_f32, hi_f32):
    lo_u = lax.bitcast_convert_type(
        lo_f32.astype(jnp.bfloat16).astype(jnp.float32), jnp.uint32)
    hi_u = lax.bitcast_convert_type(
        hi_f32.astype(jnp.bfloat16).astype(jnp.float32), jnp.uint32)
    packed = (lo_u >> 16) | (hi_u & jnp.uint32(0xFFFF0000))
    return lax.bitcast_convert_type(packed, jnp.float32)


def _unpack_pair(packed_f32):
    u = lax.bitcast_convert_type(packed_f32, jnp.uint32)
    lo = lax.bitcast_convert_type(u << 16, jnp.float32)
    hi = lax.bitcast_convert_type(u & jnp.uint32(0xFFFF0000), jnp.float32)
    return lo, hi


def _proj_body(nodes_ref, w_ref, out_ref):
    p = jnp.dot(nodes_ref[...], w_ref[...],
                preferred_element_type=jnp.float32)
    h2 = p.shape[1]
    q = h2 // 4
    pack_s = _pack_pair(p[:, 0 * q:1 * q], p[:, 1 * q:2 * q])
    pack_r = _pack_pair(p[:, 2 * q:3 * q], p[:, 3 * q:4 * q])
    out_ref[...] = jnp.concatenate([pack_s, pack_r], axis=1)


def _project_nodes(nodes, w_sr):
    n, d = nodes.shape
    two_h = w_sr.shape[1]
    bn = 2000
    grid = (n // bn,)
    return pl.pallas_call(
        _proj_body,
        grid=grid,
        in_specs=[
            pl.BlockSpec((bn, d), lambda i: (i, 0)),
            pl.BlockSpec((d, two_h), lambda i: (0, 0)),
        ],
        out_specs=pl.BlockSpec((bn, two_h // 2), lambda i: (i, 0)),
        out_shape=jax.ShapeDtypeStruct((n, two_h // 2), jnp.float32),
    )(nodes, w_sr)


_CH = 80
_NB = 4
_LA = 2


def _sc_gather(table, idx, h):
    info = plsc.get_sparse_core_info()
    nc, ns = info.num_cores, info.num_subcores
    nw = nc * ns
    total_rows = idx.shape[0]
    rows_per_w = total_rows // nw
    n_ch = rows_per_w // _CH
    n_outer = (n_ch + _NB - 1) // _NB
    dt = table.dtype
    mesh = plsc.VectorSubcoreMesh(core_axis_name="c", subcore_axis_name="s")

    @functools.partial(
        pl.kernel,
        mesh=mesh,
        out_type=jax.ShapeDtypeStruct((total_rows, h), dt),
        scratch_types=[
            pltpu.VMEM((rows_per_w,), jnp.int32),
        ] + [pltpu.VMEM((_CH, h), dt)] * _NB
          + [pltpu.SemaphoreType.DMA] * (2 * _NB),
    )
    def k(table_hbm, idx_hbm, out_hbm, idx_v, *bufsems):
        rows = bufsems[:_NB]
        gsem = bufsems[_NB:2 * _NB]
        osem = bufsems[2 * _NB:]
        wid = lax.axis_index("s") * nc + lax.axis_index("c")
        rbase = wid * rows_per_w
        pltpu.sync_copy(idx_hbm.at[pl.ds(rbase, rows_per_w)], idx_v)

        def gstart(c, b):
            pltpu.async_copy(table_hbm.at[idx_v.at[pl.ds(c * _CH, _CH)]],
                             rows[b], gsem[b])

        def ostart(c, b):
            pltpu.async_copy(rows[b],
                             out_hbm.at[pl.ds(rbase + c * _CH, _CH)],
                             osem[b])

        def owait(b):
            pltpu.make_async_copy(rows[b],
                                  out_hbm.at[pl.ds(rbase, _CH)],
                                  osem[b]).wait()

        def gwait(b):
            pltpu.make_async_copy(table_hbm.at[idx_v.at[pl.ds(0, _CH)]],
                                  rows[b], gsem[b]).wait()

        for p in range(_LA):
            gstart(p, p)

        def outer(s, carry):
            for b in range(_NB):
                c = s * _NB + b

                @pl.when(c < n_ch)
                def _(c=c, b=b):
                    gwait(b)
                    j = c + _LA
                    bj = (b + _LA) % _NB

                    @pl.when(j < n_ch)
                    def _(j=j, bj=bj):
                        @pl.when(j >= _NB)
                        def _():
                            owait(bj)
                        gstart(j, bj)

                    ostart(c, b)
            return carry

        lax.fori_loop(0, n_outer, outer, 0)
        for t in range(min(_NB, n_ch)):
            owait((n_ch - 1 - t) % _NB)

    return k(table, idx)


def _mlp_body(sg_ref, rg_ref, ea_ref, g_ref, w1e_ref, w1g_ref, b1_ref,
              w2_ref, b2_ref, out_ref):
    s_lo, s_hi = _unpack_pair(sg_ref[...])
    r_lo, r_hi = _unpack_pair(rg_ref[...])
    nodes_term = jnp.concatenate([s_lo + r_lo, s_hi + r_hi], axis=1)
    acc = nodes_term + jnp.dot(ea_ref[...], w1e_ref[...],
                               preferred_element_type=jnp.float32)
    gterm = jnp.dot(g_ref[...], w1g_ref[...],
                    preferred_element_type=jnp.float32)
    h = jnp.maximum(acc + gterm + b1_ref[...], 0.0)
    out_ref[...] = jnp.dot(h, w2_ref[...],
                           preferred_element_type=jnp.float32) + b2_ref[...]


def _edge_mlp(gathered, ea, g2d, w1e, w1g, b1, w2, b2):
    e, de = ea.shape
    h = w2.shape[0]
    dout = w2.shape[1]
    be = 800
    nblk = e // be
    return pl.pallas_call(
        _mlp_body,
        grid=(nblk,),
        in_specs=[
            pl.BlockSpec((be, h // 2), lambda i: (i, 0)),
            pl.BlockSpec((be, h // 2), lambda i: (i + nblk, 0)),
            pl.BlockSpec((be, de), lambda i: (i, 0)),
            pl.BlockSpec(g2d.shape, lambda i: (0, 0)),
            pl.BlockSpec(w1e.shape, lambda i: (0, 0)),
            pl.BlockSpec(w1g.shape, lambda i: (0, 0)),
            pl.BlockSpec(b1.shape, lambda i: (0, 0)),
            pl.BlockSpec(w2.shape, lambda i: (0, 0)),
            pl.BlockSpec(b2.shape, lambda i: (0, 0)),
        ],
        out_specs=pl.BlockSpec((be, dout), lambda i: (i, 0)),
        out_shape=jax.ShapeDtypeStruct((e, dout), jnp.float32),
    )(gathered, gathered, ea, g2d, w1e, w1g, b1, w2, b2)


def kernel(node_attributes, edge_index, edge_attributes, global_attributes,
           W1, b1, W2, b2):
    n, d = node_attributes.shape
    e, de = edge_attributes.shape
    dg = global_attributes.shape[0]
    h = W1.shape[1]

    w1e = W1[:de]
    w_sr = jnp.concatenate([W1[de:de + d], W1[de + d:de + 2 * d]], axis=1)
    w1g = W1[de + 2 * d:]

    proj = _project_nodes(node_attributes, w_sr).reshape(2 * n, h // 2)

    nslab = 5
    es = e // nslab
    src2 = 2 * edge_index[0]
    dst2 = 2 * edge_index[1] + 1
    g2d = global_attributes.reshape(1, dg)
    b1r = b1.reshape(1, h)
    b2r = b2.reshape(1, -1)
    outs = []
    for k in range(nslab):
        idx_k = jnp.concatenate([lax.dynamic_slice_in_dim(src2, k * es, es),
                                 lax.dynamic_slice_in_dim(dst2, k * es, es)])
        gathered = _sc_gather(proj, idx_k, h // 2)
        outs.append(_edge_mlp(gathered,
                              lax.dynamic_slice_in_dim(edge_attributes,
                                                       k * es, es),
                              g2d, w1e, w1g, b1r, W2, b2r))
    return jnp.concatenate(outs, axis=0)


# speedup vs baseline: 1.2295x; 1.1779x over previous
import functools

import jax
import jax.numpy as jnp
from jax import lax
from jax.experimental import pallas as pl
from jax.experimental.pallas import tpu as pltpu
from jax.experimental.pallas import tpu_sc as plsc


def _pack_pair(lo_f32, hi_f32):
    lo_u = lax.bitcast_convert_type(
        lo_f32.astype(jnp.bfloat16).astype(jnp.float32), jnp.uint32)
    hi_u = lax.bitcast_convert_type(
        hi_f32.astype(jnp.bfloat16).astype(jnp.float32), jnp.uint32)
    packed = (lo_u >> 16) | (hi_u & jnp.uint32(0xFFFF0000))
    return lax.bitcast_convert_type(packed, jnp.float32)


def _unpack_pair(packed_f32):
    u = lax.bitcast_convert_type(packed_f32, jnp.uint32)
    lo = lax.bitcast_convert_type(u << 16, jnp.float32)
    hi = lax.bitcast_convert_type(u & jnp.uint32(0xFFFF0000), jnp.float32)
    return lo, hi


def _proj_body(nodes_ref, w_ref, out_ref):
    p = jnp.dot(nodes_ref[...], w_ref[...],
                preferred_element_type=jnp.float32)
    h2 = p.shape[1]
    q = h2 // 4
    pack_s = _pack_pair(p[:, 0 * q:1 * q], p[:, 1 * q:2 * q])
    pack_r = _pack_pair(p[:, 2 * q:3 * q], p[:, 3 * q:4 * q])
    out_ref[...] = jnp.concatenate([pack_s, pack_r], axis=1)


def _project_nodes(nodes, w_sr):
    n, d = nodes.shape
    two_h = w_sr.shape[1]
    bn = 2000
    grid = (n // bn,)
    return pl.pallas_call(
        _proj_body,
        grid=grid,
        in_specs=[
            pl.BlockSpec((bn, d), lambda i: (i, 0)),
            pl.BlockSpec((d, two_h), lambda i: (0, 0)),
        ],
        out_specs=pl.BlockSpec((bn, two_h // 2), lambda i: (i, 0)),
        out_shape=jax.ShapeDtypeStruct((n, two_h // 2), jnp.float32),
    )(nodes, w_sr)


_CH = 80
_NB = 4
_LA = 2


def _sc_gather(table, idx, h):
    info = plsc.get_sparse_core_info()
    nc, ns = info.num_cores, info.num_subcores
    nw = nc * ns
    total_rows = idx.shape[0]
    rows_per_w = total_rows // nw
    n_ch = rows_per_w // _CH
    n_outer = (n_ch + _NB - 1) // _NB
    dt = table.dtype
    mesh = plsc.VectorSubcoreMesh(core_axis_name="c", subcore_axis_name="s")

    @functools.partial(
        pl.kernel,
        mesh=mesh,
        out_type=jax.ShapeDtypeStruct((total_rows, h), dt),
        scratch_types=[
            pltpu.VMEM((rows_per_w,), jnp.int32),
        ] + [pltpu.VMEM((_CH, h), dt)] * _NB
          + [pltpu.SemaphoreType.DMA] * (2 * _NB),
    )
    def k(table_hbm, idx_hbm, out_hbm, idx_v, *bufsems):
        rows = bufsems[:_NB]
        gsem = bufsems[_NB:2 * _NB]
        osem = bufsems[2 * _NB:]
        wid = lax.axis_index("s") * nc + lax.axis_index("c")
        rbase = wid * rows_per_w
        pltpu.sync_copy(idx_hbm.at[pl.ds(rbase, rows_per_w)], idx_v)

        def gstart(c, b):
            pltpu.async_copy(table_hbm.at[idx_v.at[pl.ds(c * _CH, _CH)]],
                             rows[b], gsem[b])

        def ostart(c, b):
            pltpu.async_copy(rows[b],
                             out_hbm.at[pl.ds(rbase + c * _CH, _CH)],
                             osem[b])

        def owait(b):
            pltpu.make_async_copy(rows[b],
                                  out_hbm.at[pl.ds(rbase, _CH)],
                                  osem[b]).wait()

        def gwait(b):
            pltpu.make_async_copy(table_hbm.at[idx_v.at[pl.ds(0, _CH)]],
                                  rows[b], gsem[b]).wait()

        for p in range(_LA):
            gstart(p, p)

        def outer(s, carry):
            for b in range(_NB):
                c = s * _NB + b

                @pl.when(c < n_ch)
                def _(c=c, b=b):
                    gwait(b)
                    j = c + _LA
                    bj = (b + _LA) % _NB

                    @pl.when(j < n_ch)
                    def _(j=j, bj=bj):
                        @pl.when(j >= _NB)
                        def _():
                            owait(bj)
                        gstart(j, bj)

                    ostart(c, b)
            return carry

        lax.fori_loop(0, n_outer, outer, 0)
        for t in range(min(_NB, n_ch)):
            owait((n_ch - 1 - t) % _NB)

    return k(table, idx)


def _mlp_body(buf_ref, sg_ref, rg_ref, ea_ref, g_ref, w1e_ref, w1g_ref,
              b1_ref, w2_ref, b2_ref, out_ref):
    del buf_ref
    s_lo, s_hi = _unpack_pair(sg_ref[...])
    r_lo, r_hi = _unpack_pair(rg_ref[...])
    nodes_term = jnp.concatenate([s_lo + r_lo, s_hi + r_hi], axis=1)
    acc = nodes_term + jnp.dot(ea_ref[...], w1e_ref[...],
                               preferred_element_type=jnp.float32)
    gterm = jnp.dot(g_ref[...], w1g_ref[...],
                    preferred_element_type=jnp.float32)
    h = jnp.maximum(acc + gterm + b1_ref[...], 0.0)
    out_ref[...] = jnp.dot(h, w2_ref[...],
                           preferred_element_type=jnp.float32) + b2_ref[...]


def _edge_mlp_slab(out_buf, gathered, ea, g2d, w1e, w1g, b1, w2, b2,
                   blk0, e_total):
    es, de = ea.shape
    h = w2.shape[0]
    dout = w2.shape[1]
    be = 800
    nblk = es // be
    data_specs = [
        pl.BlockSpec((be, h // 2), lambda i: (i, 0)),
        pl.BlockSpec((be, h // 2), lambda i: (i + nblk, 0)),
        pl.BlockSpec((be, de), lambda i: (i, 0)),
        pl.BlockSpec(g2d.shape, lambda i: (0, 0)),
        pl.BlockSpec(w1e.shape, lambda i: (0, 0)),
        pl.BlockSpec(w1g.shape, lambda i: (0, 0)),
        pl.BlockSpec(b1.shape, lambda i: (0, 0)),
        pl.BlockSpec(w2.shape, lambda i: (0, 0)),
        pl.BlockSpec(b2.shape, lambda i: (0, 0)),
    ]
    if out_buf is None:
        body = functools.partial(_mlp_body, None)
        in_specs = data_specs
        operands = (gathered, gathered, ea, g2d, w1e, w1g, b1, w2, b2)
        aliases = {}
    else:
        body = _mlp_body
        in_specs = [pl.BlockSpec(memory_space=pltpu.MemorySpace.HBM)]
        in_specs += data_specs
        operands = (out_buf, gathered, gathered, ea, g2d, w1e, w1g, b1, w2,
                    b2)
        aliases = {0: 0}
    return pl.pallas_call(
        body,
        grid=(nblk,),
        in_specs=in_specs,
        out_specs=pl.BlockSpec((be, dout), lambda i: (i + blk0, 0)),
        out_shape=jax.ShapeDtypeStruct((e_total, dout), jnp.float32),
        input_output_aliases=aliases,
    )(*operands)


def kernel(node_attributes, edge_index, edge_attributes, global_attributes,
           W1, b1, W2, b2):
    n, d = node_attributes.shape
    e, de = edge_attributes.shape
    dg = global_attributes.shape[0]
    h = W1.shape[1]

    w1e = W1[:de]
    w_sr = jnp.concatenate([W1[de:de + d], W1[de + d:de + 2 * d]], axis=1)
    w1g = W1[de + 2 * d:]

    proj = _project_nodes(node_attributes, w_sr).reshape(2 * n, h // 2)

    nslab = 5
    es = e // nslab
    src2 = 2 * edge_index[0]
    dst2 = 2 * edge_index[1] + 1
    g2d = global_attributes.reshape(1, dg)
    b1r = b1.reshape(1, h)
    b2r = b2.reshape(1, -1)
    blocks_per_slab = es // 800
    out = None
    for k in range(nslab):
        idx_k = jnp.concatenate([lax.dynamic_slice_in_dim(src2, k * es, es),
                                 lax.dynamic_slice_in_dim(dst2, k * es, es)])
        gathered = _sc_gather(proj, idx_k, h // 2)
        out = _edge_mlp_slab(out, gathered,
                             lax.dynamic_slice_in_dim(edge_attributes,
                                                      k * es, es),
                             g2d, w1e, w1g, b1r, W2, b2r,
                             k * blocks_per_slab, e)
    return out


# speedup vs baseline: 1.2906x; 1.0497x over previous
import functools

import jax
import jax.numpy as jnp
from jax import lax
from jax.experimental import pallas as pl
from jax.experimental.pallas import tpu as pltpu
from jax.experimental.pallas import tpu_sc as plsc


def _pack_pair(lo_f32, hi_f32):
    lo_u = lax.bitcast_convert_type(
        lo_f32.astype(jnp.bfloat16).astype(jnp.float32), jnp.uint32)
    hi_u = lax.bitcast_convert_type(
        hi_f32.astype(jnp.bfloat16).astype(jnp.float32), jnp.uint32)
    packed = (lo_u >> 16) | (hi_u & jnp.uint32(0xFFFF0000))
    return lax.bitcast_convert_type(packed, jnp.float32)


def _unpack_pair(packed_f32):
    u = lax.bitcast_convert_type(packed_f32, jnp.uint32)
    lo = lax.bitcast_convert_type(u << 16, jnp.float32)
    hi = lax.bitcast_convert_type(u & jnp.uint32(0xFFFF0000), jnp.float32)
    return lo, hi


def _proj_body(nodes_ref, w_ref, out_ref):
    p = jnp.dot(nodes_ref[...], w_ref[...],
                preferred_element_type=jnp.float32)
    h2 = p.shape[1]
    q = h2 // 4
    pack_s = _pack_pair(p[:, 0 * q:1 * q], p[:, 1 * q:2 * q])
    pack_r = _pack_pair(p[:, 2 * q:3 * q], p[:, 3 * q:4 * q])
    out_ref[...] = jnp.concatenate([pack_s, pack_r], axis=1)


def _project_nodes(nodes, w_sr):
    n, d = nodes.shape
    two_h = w_sr.shape[1]
    bn = 2000
    grid = (n // bn,)
    return pl.pallas_call(
        _proj_body,
        grid=grid,
        in_specs=[
            pl.BlockSpec((bn, d), lambda i: (i, 0)),
            pl.BlockSpec((d, two_h), lambda i: (0, 0)),
        ],
        out_specs=pl.BlockSpec((bn, two_h // 2), lambda i: (i, 0)),
        out_shape=jax.ShapeDtypeStruct((n, two_h // 2), jnp.float32),
    )(nodes, w_sr)


_CH = 80
_NB = 5
_LA = 2


def _sc_gather(table, idx, h):
    info = plsc.get_sparse_core_info()
    nc, ns = info.num_cores, info.num_subcores
    nw = nc * ns
    total_rows = idx.shape[0]
    rows_per_w = total_rows // nw
    n_ch = rows_per_w // _CH
    n_outer = (n_ch + _NB - 1) // _NB
    dt = table.dtype
    mesh = plsc.VectorSubcoreMesh(core_axis_name="c", subcore_axis_name="s")

    @functools.partial(
        pl.kernel,
        mesh=mesh,
        out_type=jax.ShapeDtypeStruct((total_rows, h), dt),
        scratch_types=[
            pltpu.VMEM((rows_per_w,), jnp.int32),
        ] + [pltpu.VMEM((_CH, h), dt)] * _NB
          + [pltpu.SemaphoreType.DMA] * (2 * _NB),
    )
    def k(table_hbm, idx_hbm, out_hbm, idx_v, *bufsems):
        rows = bufsems[:_NB]
        gsem = bufsems[_NB:2 * _NB]
        osem = bufsems[2 * _NB:]
        wid = lax.axis_index("s") * nc + lax.axis_index("c")
        rbase = wid * rows_per_w
        pltpu.sync_copy(idx_hbm.at[pl.ds(rbase, rows_per_w)], idx_v)

        def gstart(c, b):
            pltpu.async_copy(table_hbm.at[idx_v.at[pl.ds(c * _CH, _CH)]],
                             rows[b], gsem[b])

        def ostart(c, b):
            pltpu.async_copy(rows[b],
                             out_hbm.at[pl.ds(rbase + c * _CH, _CH)],
                             osem[b])

        def owait(b):
            pltpu.make_async_copy(rows[b],
                                  out_hbm.at[pl.ds(rbase, _CH)],
                                  osem[b]).wait()

        def gwait(b):
            pltpu.make_async_copy(table_hbm.at[idx_v.at[pl.ds(0, _CH)]],
                                  rows[b], gsem[b]).wait()

        for p in range(_LA):
            gstart(p, p)

        def outer(s, carry):
            for b in range(_NB):
                c = s * _NB + b

                @pl.when(c < n_ch)
                def _(c=c, b=b):
                    gwait(b)
                    j = c + _LA
                    bj = (b + _LA) % _NB

                    @pl.when(j < n_ch)
                    def _(j=j, bj=bj):
                        @pl.when(j >= _NB)
                        def _():
                            owait(bj)
                        gstart(j, bj)

                    ostart(c, b)
            return carry

        lax.fori_loop(0, n_outer, outer, 0)
        for t in range(min(_NB, n_ch)):
            owait((n_ch - 1 - t) % _NB)

    return k(table, idx)


def _mlp_body(buf_ref, sg_ref, rg_ref, ea_ref, g_ref, w1e_ref, w1g_ref,
              b1_ref, w2_ref, b2_ref, out_ref):
    del buf_ref
    s_lo, s_hi = _unpack_pair(sg_ref[...])
    r_lo, r_hi = _unpack_pair(rg_ref[...])
    nodes_term = jnp.concatenate([s_lo + r_lo, s_hi + r_hi], axis=1)
    acc = nodes_term + jnp.dot(ea_ref[...], w1e_ref[...],
                               preferred_element_type=jnp.float32)
    gterm = jnp.dot(g_ref[...], w1g_ref[...],
                    preferred_element_type=jnp.float32)
    h = jnp.maximum(acc + gterm + b1_ref[...], 0.0)
    out_ref[...] = jnp.dot(h, w2_ref[...],
                           preferred_element_type=jnp.float32) + b2_ref[...]


def _edge_mlp_slab(out_buf, gathered, ea, g2d, w1e, w1g, b1, w2, b2,
                   blk0, e_total):
    es, de = ea.shape
    h = w2.shape[0]
    dout = w2.shape[1]
    be = 1600
    nblk = es // be
    data_specs = [
        pl.BlockSpec((be, h // 2), lambda i: (i, 0)),
        pl.BlockSpec((be, h // 2), lambda i: (i + nblk, 0)),
        pl.BlockSpec((be, de), lambda i: (i, 0)),
        pl.BlockSpec(g2d.shape, lambda i: (0, 0)),
        pl.BlockSpec(w1e.shape, lambda i: (0, 0)),
        pl.BlockSpec(w1g.shape, lambda i: (0, 0)),
        pl.BlockSpec(b1.shape, lambda i: (0, 0)),
        pl.BlockSpec(w2.shape, lambda i: (0, 0)),
        pl.BlockSpec(b2.shape, lambda i: (0, 0)),
    ]
    if out_buf is None:
        body = functools.partial(_mlp_body, None)
        in_specs = data_specs
        operands = (gathered, gathered, ea, g2d, w1e, w1g, b1, w2, b2)
        aliases = {}
    else:
        body = _mlp_body
        in_specs = [pl.BlockSpec(memory_space=pltpu.MemorySpace.HBM)]
        in_specs += data_specs
        operands = (out_buf, gathered, gathered, ea, g2d, w1e, w1g, b1, w2,
                    b2)
        aliases = {0: 0}
    return pl.pallas_call(
        body,
        grid=(nblk,),
        in_specs=in_specs,
        out_specs=pl.BlockSpec((be, dout), lambda i: (i + blk0, 0)),
        out_shape=jax.ShapeDtypeStruct((e_total, dout), jnp.float32),
        input_output_aliases=aliases,
    )(*operands)


def kernel(node_attributes, edge_index, edge_attributes, global_attributes,
           W1, b1, W2, b2):
    n, d = node_attributes.shape
    e, de = edge_attributes.shape
    dg = global_attributes.shape[0]
    h = W1.shape[1]

    w1e = W1[:de]
    w_sr = jnp.concatenate([W1[de:de + d], W1[de + d:de + 2 * d]], axis=1)
    w1g = W1[de + 2 * d:]

    proj = _project_nodes(node_attributes, w_sr).reshape(2 * n, h // 2)

    nslab = 1
    es = e // nslab
    src2 = 2 * edge_index[0]
    dst2 = 2 * edge_index[1] + 1
    g2d = global_attributes.reshape(1, dg)
    b1r = b1.reshape(1, h)
    b2r = b2.reshape(1, -1)
    blocks_per_slab = es // 1600
    out = None
    for k in range(nslab):
        idx_k = jnp.concatenate([lax.dynamic_slice_in_dim(src2, k * es, es),
                                 lax.dynamic_slice_in_dim(dst2, k * es, es)])
        gathered = _sc_gather(proj, idx_k, h // 2)
        out = _edge_mlp_slab(out, gathered,
                             lax.dynamic_slice_in_dim(edge_attributes,
                                                      k * es, es),
                             g2d, w1e, w1g, b1r, W2, b2r,
                             k * blocks_per_slab, e)
    return out


# speedup vs baseline: 1.3673x; 1.0594x over previous
import functools

import jax
import jax.numpy as jnp
from jax import lax
from jax.experimental import pallas as pl
from jax.experimental.pallas import tpu as pltpu
from jax.experimental.pallas import tpu_sc as plsc


def _pack_pair(lo_f32, hi_f32):
    lo_u = lax.bitcast_convert_type(
        lo_f32.astype(jnp.bfloat16).astype(jnp.float32), jnp.uint32)
    hi_u = lax.bitcast_convert_type(
        hi_f32.astype(jnp.bfloat16).astype(jnp.float32), jnp.uint32)
    packed = (lo_u >> 16) | (hi_u & jnp.uint32(0xFFFF0000))
    return lax.bitcast_convert_type(packed, jnp.float32)


def _unpack_pair(packed_f32):
    u = lax.bitcast_convert_type(packed_f32, jnp.uint32)
    lo = lax.bitcast_convert_type(u << 16, jnp.float32)
    hi = lax.bitcast_convert_type(u & jnp.uint32(0xFFFF0000), jnp.float32)
    return lo, hi


def _proj_body(nodes_ref, w_ref, g_ref, w1g_ref, b1_ref, out_ref, b1e_ref):
    p = jnp.dot(nodes_ref[...], w_ref[...],
                preferred_element_type=jnp.float32)
    h2 = p.shape[1]
    q = h2 // 4
    pack_s = _pack_pair(p[:, 0 * q:1 * q], p[:, 1 * q:2 * q])
    pack_r = _pack_pair(p[:, 2 * q:3 * q], p[:, 3 * q:4 * q])
    out_ref[...] = jnp.concatenate([pack_s, pack_r], axis=1)
    b1e_ref[...] = b1_ref[...] + jnp.dot(
        g_ref[...], w1g_ref[...], preferred_element_type=jnp.float32)


def _project_nodes(nodes, w_sr, g2d, w1g, b1r):
    n, d = nodes.shape
    two_h = w_sr.shape[1]
    h = two_h // 2
    bn = 2000
    grid = (n // bn,)
    return pl.pallas_call(
        _proj_body,
        grid=grid,
        in_specs=[
            pl.BlockSpec((bn, d), lambda i: (i, 0)),
            pl.BlockSpec((d, two_h), lambda i: (0, 0)),
            pl.BlockSpec(g2d.shape, lambda i: (0, 0)),
            pl.BlockSpec(w1g.shape, lambda i: (0, 0)),
            pl.BlockSpec(b1r.shape, lambda i: (0, 0)),
        ],
        out_specs=[
            pl.BlockSpec((bn, two_h // 2), lambda i: (i, 0)),
            pl.BlockSpec((1, h), lambda i: (0, 0)),
        ],
        out_shape=[
            jax.ShapeDtypeStruct((n, two_h // 2), jnp.float32),
            jax.ShapeDtypeStruct((1, h), jnp.float32),
        ],
    )(nodes, w_sr, g2d, w1g, b1r)


_CH = 80
_NB = 5
_LA = 2


def _sc_gather(table, idx, h):
    info = plsc.get_sparse_core_info()
    nc, ns = info.num_cores, info.num_subcores
    nw = nc * ns
    total_rows = idx.shape[0]
    rows_per_w = total_rows // nw
    n_ch = rows_per_w // _CH
    n_outer = (n_ch + _NB - 1) // _NB
    dt = table.dtype
    mesh = plsc.VectorSubcoreMesh(core_axis_name="c", subcore_axis_name="s")

    @functools.partial(
        pl.kernel,
        mesh=mesh,
        out_type=jax.ShapeDtypeStruct((total_rows, h), dt),
        scratch_types=[
            pltpu.VMEM((rows_per_w,), jnp.int32),
        ] + [pltpu.VMEM((_CH, h), dt)] * _NB
          + [pltpu.SemaphoreType.DMA] * (2 * _NB),
    )
    def k(table_hbm, idx_hbm, out_hbm, idx_v, *bufsems):
        rows = bufsems[:_NB]
        gsem = bufsems[_NB:2 * _NB]
        osem = bufsems[2 * _NB:]
        wid = lax.axis_index("s") * nc + lax.axis_index("c")
        rbase = wid * rows_per_w
        pltpu.sync_copy(idx_hbm.at[pl.ds(rbase, rows_per_w)], idx_v)

        def gstart(c, b):
            pltpu.async_copy(table_hbm.at[idx_v.at[pl.ds(c * _CH, _CH)]],
                             rows[b], gsem[b])

        def ostart(c, b):
            pltpu.async_copy(rows[b],
                             out_hbm.at[pl.ds(rbase + c * _CH, _CH)],
                             osem[b])

        def owait(b):
            pltpu.make_async_copy(rows[b],
                                  out_hbm.at[pl.ds(rbase, _CH)],
                                  osem[b]).wait()

        def gwait(b):
            pltpu.make_async_copy(table_hbm.at[idx_v.at[pl.ds(0, _CH)]],
                                  rows[b], gsem[b]).wait()

        for p in range(_LA):
            gstart(p, p)

        def outer(s, carry):
            for b in range(_NB):
                c = s * _NB + b

                @pl.when(c < n_ch)
                def _(c=c, b=b):
                    gwait(b)
                    j = c + _LA
                    bj = (b + _LA) % _NB

                    @pl.when(j < n_ch)
                    def _(j=j, bj=bj):
                        @pl.when(j >= _NB)
                        def _():
                            owait(bj)
                        gstart(j, bj)

                    ostart(c, b)
            return carry

        lax.fori_loop(0, n_outer, outer, 0)
        for t in range(min(_NB, n_ch)):
            owait((n_ch - 1 - t) % _NB)

    return k(table, idx)


def _mlp_body(buf_ref, sg_ref, rg_ref, ea_ref, w1e_ref, b1e_ref, w2_ref,
              b2_ref, out_ref):
    del buf_ref
    s_lo, s_hi = _unpack_pair(sg_ref[...])
    r_lo, r_hi = _unpack_pair(rg_ref[...])
    t = jnp.dot(ea_ref[...], w1e_ref[...],
                preferred_element_type=jnp.float32) + b1e_ref[...]
    hh = t.shape[1] // 2
    h_lo = jnp.maximum(s_lo + r_lo + t[:, :hh], 0.0)
    h_hi = jnp.maximum(s_hi + r_hi + t[:, hh:], 0.0)
    out_ref[...] = (jnp.dot(h_lo, w2_ref[:hh, :],
                            preferred_element_type=jnp.float32) +
                    jnp.dot(h_hi, w2_ref[hh:, :],
                            preferred_element_type=jnp.float32) +
                    b2_ref[...])


def _edge_mlp_slab(out_buf, gathered, ea, w1e, b1e, w2, b2, blk0, e_total):
    es, de = ea.shape
    h = w2.shape[0]
    dout = w2.shape[1]
    be = 1600
    nblk = es // be
    data_specs = [
        pl.BlockSpec((be, h // 2), lambda i: (i, 0)),
        pl.BlockSpec((be, h // 2), lambda i: (i + nblk, 0)),
        pl.BlockSpec((be, de), lambda i: (i, 0)),
        pl.BlockSpec(w1e.shape, lambda i: (0, 0)),
        pl.BlockSpec(b1e.shape, lambda i: (0, 0)),
        pl.BlockSpec(w2.shape, lambda i: (0, 0)),
        pl.BlockSpec(b2.shape, lambda i: (0, 0)),
    ]
    if out_buf is None:
        body = functools.partial(_mlp_body, None)
        in_specs = data_specs
        operands = (gathered, gathered, ea, w1e, b1e, w2, b2)
        aliases = {}
    else:
        body = _mlp_body
        in_specs = [pl.BlockSpec(memory_space=pltpu.MemorySpace.HBM)]
        in_specs += data_specs
        operands = (out_buf, gathered, gathered, ea, w1e, b1e, w2, b2)
        aliases = {0: 0}
    return pl.pallas_call(
        body,
        grid=(nblk,),
        in_specs=in_specs,
        out_specs=pl.BlockSpec((be, dout), lambda i: (i + blk0, 0)),
        out_shape=jax.ShapeDtypeStruct((e_total, dout), jnp.float32),
        input_output_aliases=aliases,
    )(*operands)


def kernel(node_attributes, edge_index, edge_attributes, global_attributes,
           W1, b1, W2, b2):
    n, d = node_attributes.shape
    e, de = edge_attributes.shape
    dg = global_attributes.shape[0]
    h = W1.shape[1]

    w1e = W1[:de]
    w_sr = jnp.concatenate([W1[de:de + d], W1[de + d:de + 2 * d]], axis=1)
    w1g = W1[de + 2 * d:]

    g2d = global_attributes.reshape(1, dg)
    b1r = b1.reshape(1, h)

    proj, b1e = _project_nodes(node_attributes, w_sr, g2d, w1g, b1r)
    proj = proj.reshape(2 * n, h // 2)

    nslab = 1
    es = e // nslab
    src2 = 2 * edge_index[0]
    dst2 = 2 * edge_index[1] + 1
    b2r = b2.reshape(1, -1)
    blocks_per_slab = es // 1600
    out = None
    for k in range(nslab):
        idx_k = jnp.concatenate([lax.dynamic_slice_in_dim(src2, k * es, es),
                                 lax.dynamic_slice_in_dim(dst2, k * es, es)])
        gathered = _sc_gather(proj, idx_k, h // 2)
        out = _edge_mlp_slab(out, gathered,
                             lax.dynamic_slice_in_dim(edge_attributes,
                                                      k * es, es),
                             w1e, b1e, W2, b2r,
                             k * blocks_per_slab, e)
    return out


# speedup vs baseline: 1.3698x; 1.0018x over previous
import functools

import jax
import jax.numpy as jnp
from jax import lax
from jax.experimental import pallas as pl
from jax.experimental.pallas import tpu as pltpu
from jax.experimental.pallas import tpu_sc as plsc


def _pack_pair(lo_f32, hi_f32):
    lo_u = lax.bitcast_convert_type(
        lo_f32.astype(jnp.bfloat16).astype(jnp.float32), jnp.uint32)
    hi_u = lax.bitcast_convert_type(
        hi_f32.astype(jnp.bfloat16).astype(jnp.float32), jnp.uint32)
    packed = (lo_u >> 16) | (hi_u & jnp.uint32(0xFFFF0000))
    return lax.bitcast_convert_type(packed, jnp.float32)


def _unpack_pair(packed_f32):
    u = lax.bitcast_convert_type(packed_f32, jnp.uint32)
    lo = lax.bitcast_convert_type(u << 16, jnp.float32)
    hi = lax.bitcast_convert_type(u & jnp.uint32(0xFFFF0000), jnp.float32)
    return lo, hi


def _proj_body(nodes_ref, w_ref, g_ref, w1g_ref, b1_ref, out_ref, b1e_ref):
    p = jnp.dot(nodes_ref[...], w_ref[...],
                preferred_element_type=jnp.float32)
    h2 = p.shape[1]
    q = h2 // 4
    pack_s = _pack_pair(p[:, 0 * q:1 * q], p[:, 1 * q:2 * q])
    pack_r = _pack_pair(p[:, 2 * q:3 * q], p[:, 3 * q:4 * q])
    out_ref[...] = jnp.concatenate([pack_s, pack_r], axis=1)
    b1e_ref[...] = b1_ref[...] + jnp.dot(
        g_ref[...], w1g_ref[...], preferred_element_type=jnp.float32)


def _project_nodes(nodes, w_sr, g2d, w1g, b1r):
    n, d = nodes.shape
    two_h = w_sr.shape[1]
    h = two_h // 2
    bn = 2000
    grid = (n // bn,)
    return pl.pallas_call(
        _proj_body,
        grid=grid,
        in_specs=[
            pl.BlockSpec((bn, d), lambda i: (i, 0)),
            pl.BlockSpec((d, two_h), lambda i: (0, 0)),
            pl.BlockSpec(g2d.shape, lambda i: (0, 0)),
            pl.BlockSpec(w1g.shape, lambda i: (0, 0)),
            pl.BlockSpec(b1r.shape, lambda i: (0, 0)),
        ],
        out_specs=[
            pl.BlockSpec((bn, two_h // 2), lambda i: (i, 0)),
            pl.BlockSpec((1, h), lambda i: (0, 0)),
        ],
        out_shape=[
            jax.ShapeDtypeStruct((n, two_h // 2), jnp.float32),
            jax.ShapeDtypeStruct((1, h), jnp.float32),
        ],
    )(nodes, w_sr, g2d, w1g, b1r)


_CH = 128
_NB = 3
_LA = 2


def _sc_gather(table, idx, h):
    info = plsc.get_sparse_core_info()
    nc, ns = info.num_cores, info.num_subcores
    nw = nc * ns
    total_rows = idx.shape[0]
    rows_per_w = total_rows // nw
    n_ch = rows_per_w // _CH
    rem = rows_per_w - n_ch * _CH
    n_outer = (n_ch + _NB - 1) // _NB
    dt = table.dtype
    mesh = plsc.VectorSubcoreMesh(core_axis_name="c", subcore_axis_name="s")

    @functools.partial(
        pl.kernel,
        mesh=mesh,
        out_type=jax.ShapeDtypeStruct((total_rows, h), dt),
        scratch_types=[
            pltpu.VMEM((rows_per_w,), jnp.int32),
        ] + [pltpu.VMEM((_CH, h), dt)] * _NB
          + [pltpu.SemaphoreType.DMA] * (2 * _NB),
    )
    def k(table_hbm, idx_hbm, out_hbm, idx_v, *bufsems):
        rows = bufsems[:_NB]
        gsem = bufsems[_NB:2 * _NB]
        osem = bufsems[2 * _NB:]
        wid = lax.axis_index("s") * nc + lax.axis_index("c")
        rbase = wid * rows_per_w
        pltpu.sync_copy(idx_hbm.at[pl.ds(rbase, rows_per_w)], idx_v)

        def gstart(c, b):
            pltpu.async_copy(table_hbm.at[idx_v.at[pl.ds(c * _CH, _CH)]],
                             rows[b], gsem[b])

        def ostart(c, b):
            pltpu.async_copy(rows[b],
                             out_hbm.at[pl.ds(rbase + c * _CH, _CH)],
                             osem[b])

        def owait(b):
            pltpu.make_async_copy(rows[b],
                                  out_hbm.at[pl.ds(rbase, _CH)],
                                  osem[b]).wait()

        def gwait(b):
            pltpu.make_async_copy(table_hbm.at[idx_v.at[pl.ds(0, _CH)]],
                                  rows[b], gsem[b]).wait()

        for p in range(_LA):
            gstart(p, p)

        def outer(s, carry):
            for b in range(_NB):
                c = s * _NB + b

                @pl.when(c < n_ch)
                def _(c=c, b=b):
                    gwait(b)
                    j = c + _LA
                    bj = (b + _LA) % _NB

                    @pl.when(j < n_ch)
                    def _(j=j, bj=bj):
                        @pl.when(j >= _NB)
                        def _():
                            owait(bj)
                        gstart(j, bj)

                    ostart(c, b)
            return carry

        lax.fori_loop(0, n_outer, outer, 0)
        if rem:
            bt = n_ch % _NB
            if n_ch >= _NB:
                owait(bt)
            toff = n_ch * _CH
            pltpu.async_copy(table_hbm.at[idx_v.at[pl.ds(toff, rem)]],
                             rows[bt].at[pl.ds(0, rem)], gsem[bt])
            pltpu.make_async_copy(
                table_hbm.at[idx_v.at[pl.ds(toff, rem)]],
                rows[bt].at[pl.ds(0, rem)], gsem[bt]).wait()
            pltpu.async_copy(rows[bt].at[pl.ds(0, rem)],
                             out_hbm.at[pl.ds(rbase + toff, rem)], osem[bt])
        for t in range(min(_NB - (1 if rem else 0), n_ch)):
            owait((n_ch - 1 - t) % _NB)
        if rem:
            pltpu.make_async_copy(
                rows[n_ch % _NB].at[pl.ds(0, rem)],
                out_hbm.at[pl.ds(rbase, rem)], osem[n_ch % _NB]).wait()

    return k(table, idx)


def _mlp_body(buf_ref, sg_ref, rg_ref, ea_ref, w1e_ref, b1e_ref, w2_ref,
              b2_ref, out_ref):
    del buf_ref
    s_lo, s_hi = _unpack_pair(sg_ref[...])
    r_lo, r_hi = _unpack_pair(rg_ref[...])
    t = jnp.dot(ea_ref[...], w1e_ref[...],
                preferred_element_type=jnp.float32) + b1e_ref[...]
    hh = t.shape[1] // 2
    h_lo = jnp.maximum(s_lo + r_lo + t[:, :hh], 0.0)
    h_hi = jnp.maximum(s_hi + r_hi + t[:, hh:], 0.0)
    out_ref[...] = (jnp.dot(h_lo, w2_ref[:hh, :],
                            preferred_element_type=jnp.float32) +
                    jnp.dot(h_hi, w2_ref[hh:, :],
                            preferred_element_type=jnp.float32) +
                    b2_ref[...])


def _edge_mlp_slab(out_buf, gathered, ea, w1e, b1e, w2, b2, blk0, e_total):
    es, de = ea.shape
    h = w2.shape[0]
    dout = w2.shape[1]
    be = 1600
    nblk = es // be
    data_specs = [
        pl.BlockSpec((be, h // 2), lambda i: (i, 0)),
        pl.BlockSpec((be, h // 2), lambda i: (i + nblk, 0)),
        pl.BlockSpec((be, de), lambda i: (i, 0)),
        pl.BlockSpec(w1e.shape, lambda i: (0, 0)),
        pl.BlockSpec(b1e.shape, lambda i: (0, 0)),
        pl.BlockSpec(w2.shape, lambda i: (0, 0)),
        pl.BlockSpec(b2.shape, lambda i: (0, 0)),
    ]
    if out_buf is None:
        body = functools.partial(_mlp_body, None)
        in_specs = data_specs
        operands = (gathered, gathered, ea, w1e, b1e, w2, b2)
        aliases = {}
    else:
        body = _mlp_body
        in_specs = [pl.BlockSpec(memory_space=pltpu.MemorySpace.HBM)]
        in_specs += data_specs
        operands = (out_buf, gathered, gathered, ea, w1e, b1e, w2, b2)
        aliases = {0: 0}
    return pl.pallas_call(
        body,
        grid=(nblk,),
        in_specs=in_specs,
        out_specs=pl.BlockSpec((be, dout), lambda i: (i + blk0, 0)),
        out_shape=jax.ShapeDtypeStruct((e_total, dout), jnp.float32),
        input_output_aliases=aliases,
    )(*operands)


def kernel(node_attributes, edge_index, edge_attributes, global_attributes,
           W1, b1, W2, b2):
    n, d = node_attributes.shape
    e, de = edge_attributes.shape
    dg = global_attributes.shape[0]
    h = W1.shape[1]

    w1e = W1[:de]
    w_sr = jnp.concatenate([W1[de:de + d], W1[de + d:de + 2 * d]], axis=1)
    w1g = W1[de + 2 * d:]

    g2d = global_attributes.reshape(1, dg)
    b1r = b1.reshape(1, h)

    proj, b1e = _project_nodes(node_attributes, w_sr, g2d, w1g, b1r)
    proj = proj.reshape(2 * n, h // 2)

    nslab = 1
    es = e // nslab
    src2 = 2 * edge_index[0]
    dst2 = 2 * edge_index[1] + 1
    b2r = b2.reshape(1, -1)
    blocks_per_slab = es // 1600
    out = None
    for k in range(nslab):
        idx_k = jnp.concatenate([lax.dynamic_slice_in_dim(src2, k * es, es),
                                 lax.dynamic_slice_in_dim(dst2, k * es, es)])
        gathered = _sc_gather(proj, idx_k, h // 2)
        out = _edge_mlp_slab(out, gathered,
                             lax.dynamic_slice_in_dim(edge_attributes,
                                                      k * es, es),
                             w1e, b1e, W2, b2r,
                             k * blocks_per_slab, e)
    return out
